# Initial kernel scaffold; baseline (speedup 1.0000x reference)
import jax
import jax.numpy as jnp
from jax.experimental import pallas as pl


def kernel(nodes_v, flat_users, flat_ratings, segment_ids, embed_u_w, embed_i_w, embed_r_w, g1_w, g1_b, g2_w, g2_b, a1_w, a1_b, a2_w, a2_b, a3_w, a3_b):
    raise NotImplementedError("write your pallas kernel here")


# speedup vs baseline: 6.8573x; 6.8573x over previous
import functools

import jax
import jax.numpy as jnp
from jax import lax
from jax.experimental import pallas as pl
from jax.experimental.pallas import tpu as pltpu
from jax.experimental.pallas import tpu_sc as plsc

B = 16
T = 16384
D = 128
NR_PAD = 8


def _sc_info():
    try:
        info = plsc.get_sparse_core_info()
        return info.num_cores, info.num_subcores
    except Exception:
        return 2, 16


def _make_sc_gather():
    NC, NS = _sc_info()
    NW = NC * NS
    rows_per_w = T // NW
    mesh = plsc.VectorSubcoreMesh(core_axis_name="c", subcore_axis_name="s")

    @functools.partial(
        pl.kernel,
        mesh=mesh,
        out_type=[
            jax.ShapeDtypeStruct((T, D), jnp.float32),
            jax.ShapeDtypeStruct((B, D), jnp.float32),
        ],
        scratch_types=[
            pltpu.VMEM((rows_per_w,), jnp.int32),
            pltpu.VMEM((rows_per_w, D), jnp.float32),
            pltpu.VMEM((B,), jnp.int32),
            pltpu.VMEM((B, D), jnp.float32),
            pltpu.SemaphoreType.DMA,
        ],
    )
    def sc_gather(u_table, u_idx, i_table, v_idx, pt_out, qj_out,
                  idx_v, rows_v, vidx_v, vrows_v, sem):
        wid = lax.axis_index("s") * NC + lax.axis_index("c")
        base = wid * rows_per_w
        pltpu.sync_copy(u_idx.at[pl.ds(base, rows_per_w)], idx_v)
        pltpu.async_copy(u_table.at[idx_v], rows_v, sem).wait()
        pltpu.sync_copy(rows_v, pt_out.at[pl.ds(base, rows_per_w)])

        @pl.when(wid == 0)
        def _():
            pltpu.sync_copy(v_idx, vidx_v)
            pltpu.async_copy(i_table.at[vidx_v], vrows_v, sem).wait()
            pltpu.sync_copy(vrows_v, qj_out)

    return sc_gather


def _dot_t(x, w):
    return lax.dot_general(x, w, (((1,), (1,)), ((), ())),
                           preferred_element_type=jnp.float32)


def _dot(x, w):
    return lax.dot_general(x, w, (((1,), (0,)), ((), ())),
                           preferred_element_type=jnp.float32)


def _tc_body(pt_ref, qj_ref, seg_ref, rat_ref, er_ref, g1_ref, g2_ref,
             a1_ref, a2_ref, a3_ref, g1b_ref, g2b_ref, a1b_ref, a2b_ref,
             a3b_ref, z_ref):
    pt = pt_ref[...]
    seg = seg_ref[...]
    rat = rat_ref[...]
    onehot_s = (seg == lax.broadcasted_iota(jnp.int32, (1, B), 1)
                ).astype(jnp.float32)
    onehot_r = (rat == lax.broadcasted_iota(jnp.int32, (1, NR_PAD), 1)
                ).astype(jnp.float32)

    g1 = g1_ref[...]
    g1A = g1[:, :D]
    g1B = g1[:, D:]
    a1 = a1_ref[...]
    a1A = a1[:, :D]
    a1B = a1[:, D:]

    er_proj = _dot_t(er_ref[...], g1B)
    qj_proj = _dot_t(qj_ref[...], a1B)

    h = jnp.maximum(_dot_t(pt, g1A) + _dot(onehot_r, er_proj)
                    + g1b_ref[...], 0.0)
    fjt = jnp.maximum(_dot_t(h, g2_ref[...]) + g2b_ref[...], 0.0)

    a = jnp.maximum(_dot_t(fjt, a1A) + _dot(onehot_s, qj_proj)
                    + a1b_ref[...], 0.0)
    a = jnp.maximum(_dot_t(a, a2_ref[...]) + a2b_ref[...], 0.0)
    a3rep = jnp.broadcast_to(a3_ref[...], (B, D))
    smat0 = _dot_t(a, a3rep) + a3b_ref[0, 0]

    smat = jnp.where(onehot_s > 0.0, smat0, -1e30)
    m = jnp.max(smat, axis=0, keepdims=True)
    emat = jnp.exp(smat - m) * onehot_s
    denom = jnp.sum(emat, axis=0, keepdims=True)
    w = emat / jnp.maximum(denom, 1e-30)

    z_ref[...] = lax.dot_general(w, fjt, (((0,), (0,)), ((), ())),
                                 preferred_element_type=jnp.float32)


def kernel(nodes_v, flat_users, flat_ratings, segment_ids, embed_u_w,
           embed_i_w, embed_r_w, g1_w, g1_b, g2_w, g2_b, a1_w, a1_b,
           a2_w, a2_b, a3_w, a3_b):
    sc_gather = _make_sc_gather()
    pt, qj = sc_gather(embed_u_w, flat_users, embed_i_w, nodes_v)

    er_pad = jnp.zeros((NR_PAD, D), jnp.float32).at[:5].set(embed_r_w)
    seg2 = segment_ids.reshape(T, 1)
    rat2 = flat_ratings.reshape(T, 1)

    z = pl.pallas_call(
        _tc_body,
        out_shape=jax.ShapeDtypeStruct((B, D), jnp.float32),
    )(pt, qj, seg2, rat2, er_pad, g1_w, g2_w, a1_w, a2_w, a3_w,
      g1_b.reshape(1, D), g2_b.reshape(1, D), a1_b.reshape(1, D),
      a2_b.reshape(1, D), a3_b.reshape(1, 1))
    return z


# speedup vs baseline: 7.7873x; 1.1356x over previous
import functools

import jax
import jax.numpy as jnp
from jax import lax
from jax.experimental import pallas as pl
from jax.experimental.pallas import tpu as pltpu
from jax.experimental.pallas import tpu_sc as plsc

B = 16
T = 16384
D = 128
NR_PAD = 8
TB = 2048
NB = T // TB


def _sc_info():
    try:
        info = plsc.get_sparse_core_info()
        return info.num_cores, info.num_subcores
    except Exception:
        return 2, 16


def _make_sc_gather():
    NC, NS = _sc_info()
    NW = NC * NS
    rows_per_w = T // NW
    mesh = plsc.VectorSubcoreMesh(core_axis_name="c", subcore_axis_name="s")

    @functools.partial(
        pl.kernel,
        mesh=mesh,
        out_type=[
            jax.ShapeDtypeStruct((T, D), jnp.float32),
            jax.ShapeDtypeStruct((B, D), jnp.float32),
        ],
        scratch_types=[
            pltpu.VMEM((rows_per_w,), jnp.int32),
            pltpu.VMEM((rows_per_w, D), jnp.float32),
            pltpu.VMEM((B,), jnp.int32),
            pltpu.VMEM((B, D), jnp.float32),
            pltpu.SemaphoreType.DMA,
        ],
    )
    def sc_gather(u_table, u_idx, i_table, v_idx, pt_out, qj_out,
                  idx_v, rows_v, vidx_v, vrows_v, sem):
        wid = lax.axis_index("s") * NC + lax.axis_index("c")
        base = wid * rows_per_w
        pltpu.sync_copy(u_idx.at[pl.ds(base, rows_per_w)], idx_v)
        pltpu.async_copy(u_table.at[idx_v], rows_v, sem).wait()
        pltpu.sync_copy(rows_v, pt_out.at[pl.ds(base, rows_per_w)])

        @pl.when(wid == 0)
        def _():
            pltpu.sync_copy(v_idx, vidx_v)
            pltpu.async_copy(i_table.at[vidx_v], vrows_v, sem).wait()
            pltpu.sync_copy(vrows_v, qj_out)

    return sc_gather


def _dot_t(x, w):
    return lax.dot_general(x, w, (((1,), (1,)), ((), ())),
                           preferred_element_type=jnp.float32)


def _bdot_t(x, w):
    return lax.dot_general(x.astype(jnp.bfloat16), w.astype(jnp.bfloat16),
                           (((1,), (1,)), ((), ())),
                           preferred_element_type=jnp.float32)


def _dot(x, w):
    return lax.dot_general(x, w, (((1,), (0,)), ((), ())),
                           preferred_element_type=jnp.float32)


def _tc_body(pt_ref, qj_ref, seg_ref, rat_ref, er_ref, g1_ref, g2_ref,
             a1_ref, a2_ref, a3_ref, g1b_ref, g2b_ref, a1b_ref, a2b_ref,
             a3b_ref, z_ref, m_sc, l_sc, zacc_sc):
    i = pl.program_id(0)

    @pl.when(i == 0)
    def _():
        m_sc[...] = jnp.full((1, B), -1e30, jnp.float32)
        l_sc[...] = jnp.zeros((1, B), jnp.float32)
        zacc_sc[...] = jnp.zeros((B, D), jnp.float32)

    pt = pt_ref[...]
    seg = seg_ref[...]
    rat = rat_ref[...]
    onehot_s = (seg == lax.broadcasted_iota(jnp.int32, (1, B), 1)
                ).astype(jnp.float32)
    onehot_r = (rat == lax.broadcasted_iota(jnp.int32, (1, NR_PAD), 1)
                ).astype(jnp.float32)

    g1 = g1_ref[...]
    a1 = a1_ref[...]

    er_proj = _dot_t(er_ref[...], g1[:, D:])
    qj_proj = _dot_t(qj_ref[...], a1[:, D:])

    h = jnp.maximum(_bdot_t(pt, g1[:, :D]) + _dot(onehot_r, er_proj)
                    + g1b_ref[...], 0.0)
    fjt = jnp.maximum(_bdot_t(h, g2_ref[...]) + g2b_ref[...], 0.0)

    a = jnp.maximum(_bdot_t(fjt, a1[:, :D]) + _dot(onehot_s, qj_proj)
                    + a1b_ref[...], 0.0)
    a = jnp.maximum(_bdot_t(a, a2_ref[...]) + a2b_ref[...], 0.0)
    a3rep = jnp.broadcast_to(a3_ref[...], (B, D))
    smat0 = _bdot_t(a, a3rep) + a3b_ref[0, 0]

    smat = jnp.where(onehot_s > 0.0, smat0, -1e30)
    m_old = m_sc[...]
    m_new = jnp.maximum(m_old, jnp.max(smat, axis=0, keepdims=True))
    scale = jnp.exp(m_old - m_new)
    e = jnp.exp(smat - m_new) * onehot_s
    l_sc[...] = l_sc[...] * scale + jnp.sum(e, axis=0, keepdims=True)
    m_sc[...] = m_new

    eye = (lax.broadcasted_iota(jnp.int32, (B, B), 0)
           == lax.broadcasted_iota(jnp.int32, (B, B), 1))
    dscale = jnp.where(eye, jnp.broadcast_to(scale, (B, B)), 0.0)
    zacc_sc[...] = (_dot(dscale, zacc_sc[...])
                    + lax.dot_general(e, fjt, (((0,), (0,)), ((), ())),
                                      preferred_element_type=jnp.float32))

    @pl.when(i == NB - 1)
    def _():
        recip = 1.0 / jnp.maximum(l_sc[...], 1e-30)
        drec = jnp.where(eye, jnp.broadcast_to(recip, (B, B)), 0.0)
        z_ref[...] = _dot(drec, zacc_sc[...])


def _tc_compute(pt, qj, seg2, rat2, er_pad, g1_w, g2_w, a1_w, a2_w, a3_w,
                g1_b, g2_b, a1_b, a2_b, a3_b):
    full = lambda s: pl.BlockSpec(s, lambda i: (0, 0))
    return pl.pallas_call(
        _tc_body,
        grid=(NB,),
        in_specs=[
            pl.BlockSpec((TB, D), lambda i: (i, 0)),
            full((B, D)),
            pl.BlockSpec((TB, 1), lambda i: (i, 0)),
            pl.BlockSpec((TB, 1), lambda i: (i, 0)),
            full((NR_PAD, D)),
            full((D, 2 * D)),
            full((D, D)),
            full((D, 2 * D)),
            full((D, D)),
            full((1, D)),
            full((1, D)),
            full((1, D)),
            full((1, D)),
            full((1, D)),
            full((1, 1)),
        ],
        out_specs=pl.BlockSpec((B, D), lambda i: (0, 0)),
        out_shape=jax.ShapeDtypeStruct((B, D), jnp.float32),
        scratch_shapes=[
            pltpu.VMEM((1, B), jnp.float32),
            pltpu.VMEM((1, B), jnp.float32),
            pltpu.VMEM((B, D), jnp.float32),
        ],
    )(pt, qj, seg2, rat2, er_pad, g1_w, g2_w, a1_w, a2_w, a3_w,
      g1_b, g2_b, a1_b, a2_b, a3_b)


def kernel(nodes_v, flat_users, flat_ratings, segment_ids, embed_u_w,
           embed_i_w, embed_r_w, g1_w, g1_b, g2_w, g2_b, a1_w, a1_b,
           a2_w, a2_b, a3_w, a3_b):
    sc_gather = _make_sc_gather()
    pt, qj = sc_gather(embed_u_w, flat_users, embed_i_w, nodes_v)

    er_pad = jnp.zeros((NR_PAD, D), jnp.float32).at[:5].set(embed_r_w)
    seg2 = segment_ids.reshape(T, 1)
    rat2 = flat_ratings.reshape(T, 1)

    return _tc_compute(pt, qj, seg2, rat2, er_pad, g1_w, g2_w, a1_w, a2_w,
                       a3_w, g1_b.reshape(1, D), g2_b.reshape(1, D),
                       a1_b.reshape(1, D), a2_b.reshape(1, D),
                       a3_b.reshape(1, 1))
